# Initial kernel scaffold; baseline (speedup 1.0000x reference)
import functools

import jax
import jax.numpy as jnp
from jax import lax
from jax.experimental import pallas as pl
from jax.experimental.pallas import tpu as pltpu
from jax.experimental.pallas import tpu_sc as plsc

BATCH = 16384
HIST = 50
EMBED = 32

NC = 2
NS = 16
NW = NC * NS

B = BATCH * HIST
BPW = B // NW
CHUNK = 128
NCH = BPW // CHUNK
K = 4
ROWS = CHUNK * K
NOUT = NCH // K

_mesh = plsc.VectorSubcoreMesh(
    core_axis_name="c", subcore_axis_name="s", num_cores=NC, num_subcores=NS
)


@functools.partial(
    pl.kernel,
    mesh=_mesh,
    out_type=jax.ShapeDtypeStruct((B, EMBED), jnp.float32),
    scratch_types=[
        pltpu.VMEM((NCH, CHUNK), jnp.int32),
        pltpu.VMEM((ROWS, EMBED), jnp.float32),
        pltpu.SemaphoreType.DMA,
    ],
)
def _emb_kernel(idx_hbm, table_hbm, out_hbm, idx_v, rows_v, gsem):
    wid = lax.axis_index("s") * NC + lax.axis_index("c")
    base = wid * BPW

    pltpu.sync_copy(idx_hbm.at[wid], idx_v)

    @pl.loop(0, NOUT)
    def _outer(o):
        copies = []
        for j in range(K):
            copies.append(
                pltpu.async_copy(
                    table_hbm.at[idx_v.at[o * K + j]],
                    rows_v.at[pl.ds(j * CHUNK, CHUNK)],
                    gsem,
                )
            )
        for c in copies:
            c.wait()
        pltpu.sync_copy(rows_v, out_hbm.at[pl.ds(base + o * ROWS, ROWS)])


def kernel(indices, table):
    idx = indices.reshape(NW, NCH, CHUNK).astype(jnp.int32)
    out = _emb_kernel(idx, table)
    return out.reshape(BATCH, HIST, EMBED)


# speedup vs baseline: 1.0899x; 1.0899x over previous
import functools

import jax
import jax.numpy as jnp
from jax import lax
from jax.experimental import pallas as pl
from jax.experimental.pallas import tpu as pltpu
from jax.experimental.pallas import tpu_sc as plsc

BATCH = 16384
HIST = 50
EMBED = 32

NC = 2
NS = 16
NW = NC * NS

B = BATCH * HIST
BPW = B // NW
CHUNK = 128
NCH = BPW // CHUNK
K = 4
ROWS = CHUNK * K
NOUT = NCH // K

_mesh = plsc.VectorSubcoreMesh(
    core_axis_name="c", subcore_axis_name="s", num_cores=NC, num_subcores=NS
)


@functools.partial(
    pl.kernel,
    mesh=_mesh,
    compiler_params=pltpu.CompilerParams(use_tc_tiling_on_sc=False),
    out_type=jax.ShapeDtypeStruct((B, EMBED), jnp.float32),
    scratch_types=[
        pltpu.VMEM((NCH, CHUNK), jnp.int32),
        pltpu.VMEM((ROWS, EMBED), jnp.float32),
        pltpu.SemaphoreType.DMA,
    ],
)
def _emb_kernel(idx_hbm, table_hbm, out_hbm, idx_v, rows_v, gsem):
    wid = lax.axis_index("s") * NC + lax.axis_index("c")
    base = wid * BPW

    pltpu.sync_copy(idx_hbm.at[wid], idx_v)

    @pl.loop(0, NOUT)
    def _outer(o):
        copies = []
        for j in range(K):
            copies.append(
                pltpu.async_copy(
                    table_hbm.at[idx_v.at[o * K + j]],
                    rows_v.at[pl.ds(j * CHUNK, CHUNK)],
                    gsem,
                )
            )
        for c in copies:
            c.wait()
        pltpu.sync_copy(rows_v, out_hbm.at[pl.ds(base + o * ROWS, ROWS)])


def kernel(indices, table):
    idx = indices.reshape(NW, NCH, CHUNK).astype(jnp.int32)
    out = _emb_kernel(idx, table)
    return out.reshape(BATCH, HIST, EMBED)
